# baseline (speedup 1.0000x reference)
import functools

import jax
import jax.numpy as jnp
from jax import lax
from jax.experimental import pallas as pl
from jax.experimental.pallas import tpu as pltpu
from jax.experimental.pallas import tpu_sc as plsc

_LANES = 16
_NUM_WORKERS = 32


@functools.cache
def _make_adj_builder(n_nodes: int, e_pad: int):
    rows_per_w = n_nodes // _NUM_WORKERS
    words_per_w = rows_per_w * n_nodes
    n_groups = e_pad // _LANES
    mesh = plsc.VectorSubcoreMesh(core_axis_name="c", subcore_axis_name="s")

    @functools.partial(
        pl.kernel,
        out_type=jax.ShapeDtypeStruct((n_nodes * n_nodes,), jnp.float32),
        mesh=mesh,
        compiler_params=pltpu.CompilerParams(needs_layout_passes=False),
        scratch_types=[
            pltpu.VMEM((2, e_pad), jnp.int32),
            pltpu.VMEM((words_per_w,), jnp.float32),
            pltpu.SemaphoreType.DMA,
        ],
    )
    def build(ei_hbm, out_hbm, ei_v, buf, sem):
        wid = lax.axis_index("s") * 2 + lax.axis_index("c")
        row_lo = wid * rows_per_w
        row_hi = row_lo + rows_per_w
        ei_cp = pltpu.async_copy(ei_hbm, ei_v, sem)

        zeros16 = jnp.zeros((_LANES,), jnp.float32)
        for i in range(words_per_w // _LANES):
            buf[pl.ds(i * _LANES, _LANES)] = zeros16
        ei_cp.wait()

        ones16 = jnp.full((_LANES,), 1.0, jnp.float32)
        for g in range(n_groups):
            s = ei_v[0, pl.ds(g * _LANES, _LANES)]
            d = ei_v[1, pl.ds(g * _LANES, _LANES)]
            mask = (d >= row_lo) & (d < row_hi)
            local = (d - row_lo) * n_nodes + s
            local = jnp.where(mask, local, 0)
            plsc.store_scatter(buf, [local], ones16, mask=mask)

        pltpu.sync_copy(buf, out_hbm.at[pl.ds(wid * words_per_w, words_per_w)])

    return build


def _build_abin(edge_index, n_nodes):
    ei = edge_index.astype(jnp.int32)
    e = ei.shape[1]
    e_pad = ((e + _LANES - 1) // _LANES) * _LANES
    if e_pad != e:
        pad = jnp.full((2, e_pad - e), n_nodes, jnp.int32)
        ei = jnp.concatenate([ei, pad], axis=1)
    flat = _make_adj_builder(n_nodes, e_pad)(ei)
    return flat.reshape(n_nodes, n_nodes)


@functools.cache
def _make_gcn(bsz: int, ch: int, hw: int):
    n_nodes = bsz * hw
    dn = (((0,), (0,)), ((), ()))

    def body(x_ref, a_ref, w1_ref, b1_ref, w2_ref, b2_ref, w3_ref, b3_ref,
             out_ref):
        a = a_ref[...]
        deg = jnp.sum(a, axis=1, keepdims=True)
        dinv = jnp.where(deg > 0, lax.rsqrt(deg), 0.0)

        def agg(hh, b_row):
            return dinv * jnp.dot(a, hh * dinv,
                                  preferred_element_type=jnp.float32) + b_row

        w1 = w1_ref[...]
        h = jnp.concatenate(
            [lax.dot_general(x_ref[b], w1, dn,
                             preferred_element_type=jnp.float32)
             for b in range(bsz)], axis=0)
        h = jnp.maximum(agg(h, b1_ref[...]), 0.0)
        h = jnp.maximum(
            agg(jnp.dot(h, w2_ref[...], preferred_element_type=jnp.float32),
                b2_ref[...]), 0.0)
        out_ref[...] = agg(
            jnp.dot(h, w3_ref[...], preferred_element_type=jnp.float32),
            b3_ref[...])

    return pl.pallas_call(
        body,
        out_shape=jax.ShapeDtypeStruct((n_nodes, ch), jnp.float32),
    )


def kernel(x, W1, b1, W2, b2, W3, b3, edge_index):
    bsz, ch, hgt, wid = x.shape
    hw = hgt * wid
    n_nodes = bsz * hw
    abin = _build_abin(edge_index, n_nodes)
    x2 = x.reshape(bsz, ch, hw)
    h = _make_gcn(bsz, ch, hw)(
        x2, abin,
        W1, b1.reshape(1, ch),
        W2, b2.reshape(1, ch),
        W3, b3.reshape(1, ch))
    return h.reshape(bsz, ch, hgt, wid)


# speedup vs baseline: 1.0031x; 1.0031x over previous
import functools

import jax
import jax.numpy as jnp
from jax import lax
from jax.experimental import pallas as pl
from jax.experimental.pallas import tpu as pltpu
from jax.experimental.pallas import tpu_sc as plsc

_LANES = 16
_NUM_WORKERS = 32


@functools.cache
def _make_adj_builder(n_nodes: int, e_pad: int):
    rows_per_w = n_nodes // _NUM_WORKERS
    words_per_w = rows_per_w * n_nodes
    n_groups = e_pad // _LANES
    mesh = plsc.VectorSubcoreMesh(core_axis_name="c", subcore_axis_name="s")

    @functools.partial(
        pl.kernel,
        out_type=jax.ShapeDtypeStruct((n_nodes * n_nodes,), jnp.float32),
        mesh=mesh,
        compiler_params=pltpu.CompilerParams(needs_layout_passes=False),
        scratch_types=[
            pltpu.VMEM((2, e_pad), jnp.int32),
            pltpu.VMEM((words_per_w,), jnp.float32),
            pltpu.SemaphoreType.DMA,
        ],
    )
    def build(ei_hbm, out_hbm, ei_v, buf, sem):
        wid = lax.axis_index("s") * 2 + lax.axis_index("c")
        row_lo = wid * rows_per_w
        row_hi = row_lo + rows_per_w
        ei_cp = pltpu.async_copy(ei_hbm, ei_v, sem)

        zeros16 = jnp.zeros((_LANES,), jnp.float32)
        for i in range(words_per_w // _LANES):
            buf[pl.ds(i * _LANES, _LANES)] = zeros16
        ei_cp.wait()

        ones16 = jnp.full((_LANES,), 1.0, jnp.float32)
        for g in range(n_groups):
            s = ei_v[0, pl.ds(g * _LANES, _LANES)]
            d = ei_v[1, pl.ds(g * _LANES, _LANES)]
            mask = (d >= row_lo) & (d < row_hi)
            local = (d - row_lo) * n_nodes + s
            local = jnp.where(mask, local, 0)
            plsc.store_scatter(buf, [local], ones16, mask=mask)

        pltpu.sync_copy(buf, out_hbm.at[pl.ds(wid * words_per_w, words_per_w)])

    return build


def _build_abin(edge_index, n_nodes):
    ei = edge_index.astype(jnp.int32)
    e = ei.shape[1]
    e_pad = ((e + _LANES - 1) // _LANES) * _LANES
    if e_pad != e:
        pad = jnp.full((2, e_pad - e), n_nodes, jnp.int32)
        ei = jnp.concatenate([ei, pad], axis=1)
    flat = _make_adj_builder(n_nodes, e_pad)(ei)
    return flat.reshape(n_nodes, n_nodes)


@functools.cache
def _make_l1(bsz: int, ch: int, hw: int):
    n_nodes = bsz * hw
    dn = (((0,), (0,)), ((), ()))

    def body(x_ref, w1_ref, out_ref):
        w1 = w1_ref[...]
        out_ref[...] = jnp.concatenate(
            [lax.dot_general(x_ref[b], w1, dn,
                             preferred_element_type=jnp.float32)
             for b in range(bsz)], axis=0)

    return pl.pallas_call(
        body, out_shape=jax.ShapeDtypeStruct((n_nodes, ch), jnp.float32))


@functools.cache
def _make_rest(n_nodes: int, ch: int):
    def body(h1_ref, a_ref, b1_ref, w2_ref, b2_ref, w3_ref, b3_ref, out_ref):
        a = a_ref[...]
        deg = jnp.sum(a, axis=1, keepdims=True)
        dinv = jnp.where(deg > 0, lax.rsqrt(deg), 0.0)

        def agg(hh, b_row):
            return dinv * jnp.dot(a, hh * dinv,
                                  preferred_element_type=jnp.float32) + b_row

        h = jnp.maximum(agg(h1_ref[...], b1_ref[...]), 0.0)
        h = jnp.maximum(
            agg(jnp.dot(h, w2_ref[...], preferred_element_type=jnp.float32),
                b2_ref[...]), 0.0)
        out_ref[...] = agg(
            jnp.dot(h, w3_ref[...], preferred_element_type=jnp.float32),
            b3_ref[...])

    return pl.pallas_call(
        body, out_shape=jax.ShapeDtypeStruct((n_nodes, ch), jnp.float32))


def kernel(x, W1, b1, W2, b2, W3, b3, edge_index):
    bsz, ch, hgt, wid = x.shape
    hw = hgt * wid
    n_nodes = bsz * hw
    abin = _build_abin(edge_index, n_nodes)
    x2 = x.reshape(bsz, ch, hw)
    h1 = _make_l1(bsz, ch, hw)(x2, W1)
    h = _make_rest(n_nodes, ch)(
        h1, abin, b1.reshape(1, ch),
        W2, b2.reshape(1, ch), W3, b3.reshape(1, ch))
    return h.reshape(bsz, ch, hgt, wid)


# speedup vs baseline: 1.0618x; 1.0585x over previous
import functools

import jax
import jax.numpy as jnp
from jax import lax
from jax.experimental import pallas as pl
from jax.experimental.pallas import tpu as pltpu
from jax.experimental.pallas import tpu_sc as plsc

_LANES = 16
_NUM_WORKERS = 32


@functools.cache
def _make_adj_builder(n_nodes: int, e_pad: int):
    rows_per_w = n_nodes // _NUM_WORKERS
    words_per_w = rows_per_w * n_nodes
    n_groups = e_pad // _LANES
    mesh = plsc.VectorSubcoreMesh(core_axis_name="c", subcore_axis_name="s")

    @functools.partial(
        pl.kernel,
        out_type=jax.ShapeDtypeStruct((n_nodes * n_nodes,), jnp.float32),
        mesh=mesh,
        compiler_params=pltpu.CompilerParams(needs_layout_passes=False),
        scratch_types=[
            pltpu.VMEM((2, e_pad), jnp.int32),
            pltpu.VMEM((words_per_w,), jnp.float32),
            pltpu.SemaphoreType.DMA,
        ],
    )
    def build(ei_hbm, out_hbm, ei_v, buf, sem):
        wid = lax.axis_index("s") * 2 + lax.axis_index("c")
        row_lo = wid * rows_per_w
        row_hi = row_lo + rows_per_w
        ei_cp = pltpu.async_copy(ei_hbm, ei_v, sem)

        zeros16 = jnp.zeros((_LANES,), jnp.float32)

        @plsc.parallel_loop(0, words_per_w, step=_LANES, unroll=8)
        def _zero(i):
            buf[pl.ds(i, _LANES)] = zeros16

        ei_cp.wait()

        ones16 = jnp.full((_LANES,), 1.0, jnp.float32)

        @plsc.parallel_loop(0, n_groups * _LANES, step=_LANES, unroll=4)
        def _scatter(g):
            s = ei_v[0, pl.ds(g, _LANES)]
            d = ei_v[1, pl.ds(g, _LANES)]
            mask = (d >= row_lo) & (d < row_hi)
            local = (d - row_lo) * n_nodes + s
            local = jnp.where(mask, local, 0)
            plsc.store_scatter(buf, [local], ones16, mask=mask)

        pltpu.sync_copy(buf, out_hbm.at[pl.ds(wid * words_per_w, words_per_w)])

    return build


def _build_abin(edge_index, n_nodes):
    ei = edge_index.astype(jnp.int32)
    e = ei.shape[1]
    e_pad = ((e + _LANES - 1) // _LANES) * _LANES
    if e_pad != e:
        pad = jnp.full((2, e_pad - e), n_nodes, jnp.int32)
        ei = jnp.concatenate([ei, pad], axis=1)
    flat = _make_adj_builder(n_nodes, e_pad)(ei)
    return flat.reshape(n_nodes, n_nodes)


@functools.cache
def _make_l1(bsz: int, ch: int, hw: int):
    n_nodes = bsz * hw
    dn = (((0,), (0,)), ((), ()))

    def body(x_ref, w1_ref, out_ref):
        w1 = w1_ref[...]
        out_ref[...] = jnp.concatenate(
            [lax.dot_general(x_ref[b], w1, dn,
                             preferred_element_type=jnp.float32)
             for b in range(bsz)], axis=0)

    return pl.pallas_call(
        body, out_shape=jax.ShapeDtypeStruct((n_nodes, ch), jnp.float32))


@functools.cache
def _make_rest(n_nodes: int, ch: int):
    def body(h1_ref, a_ref, b1_ref, w2_ref, b2_ref, w3_ref, b3_ref, out_ref):
        a = a_ref[...]
        deg = jnp.sum(a, axis=1, keepdims=True)
        dinv = jnp.where(deg > 0, lax.rsqrt(deg), 0.0)

        def agg(hh, b_row):
            return dinv * jnp.dot(a, hh * dinv,
                                  preferred_element_type=jnp.float32) + b_row

        h = jnp.maximum(agg(h1_ref[...], b1_ref[...]), 0.0)
        h = jnp.maximum(
            agg(jnp.dot(h, w2_ref[...], preferred_element_type=jnp.float32),
                b2_ref[...]), 0.0)
        out_ref[...] = agg(
            jnp.dot(h, w3_ref[...], preferred_element_type=jnp.float32),
            b3_ref[...])

    return pl.pallas_call(
        body, out_shape=jax.ShapeDtypeStruct((n_nodes, ch), jnp.float32))


def kernel(x, W1, b1, W2, b2, W3, b3, edge_index):
    bsz, ch, hgt, wid = x.shape
    hw = hgt * wid
    n_nodes = bsz * hw
    abin = _build_abin(edge_index, n_nodes)
    x2 = x.reshape(bsz, ch, hw)
    h1 = _make_l1(bsz, ch, hw)(x2, W1)
    h = _make_rest(n_nodes, ch)(
        h1, abin, b1.reshape(1, ch),
        W2, b2.reshape(1, ch), W3, b3.reshape(1, ch))
    return h.reshape(bsz, ch, hgt, wid)
